# baseline (speedup 1.0000x reference)
import functools

import jax
import jax.numpy as jnp
from jax.experimental import pallas as pl
from jax.experimental.pallas import tpu as pltpu

_KEEP_RATIO = 0.5
_ROWS1 = 1024
_ROWS2 = 1024
_CHUNK = 256


def _fused_kernel(keep, nsteps1, row_ref, col_ref, x1_ref, x2_ref, coef_ref,
                  o_ref, acc_ref, mask_ref, ab_ref, loss_ref):
    i = pl.program_id(0)
    D = row_ref.shape[1]
    deg = coef_ref.shape[1]

    @pl.when(i == 0)
    def _mask_and_init():
        row = row_ref[...]
        kidx0 = jax.lax.broadcasted_iota(jnp.int32, (_CHUNK, D), 0)
        qidx = jax.lax.broadcasted_iota(jnp.int32, (_CHUNK, D), 1)
        rank = jnp.zeros((1, D), jnp.float32)
        for c in range(D // _CHUNK):
            col = col_ref[pl.ds(c * _CHUNK, _CHUNK), :]
            kidx = kidx0 + c * _CHUNK
            beat = jnp.logical_or(
                col > row,
                jnp.logical_and(col == row, kidx < qidx))
            rank = rank + jnp.sum(jnp.where(beat, 1.0, 0.0),
                                  axis=0, keepdims=True)
        m = rank < keep
        mask_ref[...] = jnp.where(m, 1.0, 0.0)
        for k in range(deg):
            ab_ref[k:k + 1, :] = jnp.where(
                m, coef_ref[0, k], 1.0 if k == 0 else 0.0)
        acc_ref[...] = jnp.zeros((1, D), jnp.float32)

    @pl.when(i < nsteps1)
    def _phase1():
        xb = x1_ref[...]
        acc_ref[...] = acc_ref[...] + jnp.sum(xb * xb, axis=0, keepdims=True)

    @pl.when(i == nsteps1 - 1)
    def _loss():
        loss_ref[0, 0] = 1e-6 * jnp.sqrt(
            jnp.sum(acc_ref[...] * (1.0 - mask_ref[...])))

    @pl.when(i >= nsteps1)
    def _phase2():
        x = x2_ref[...]
        p = ab_ref[deg - 1:deg, :] * x
        for k in range(deg - 2, -1, -1):
            p = (p + ab_ref[k:k + 1, :]) * x
        o_ref[...] = p + loss_ref[0, 0]


def kernel(x, coeffs, importance):
    B, T, D = x.shape
    keep = max(1, int(D * _KEEP_RATIO))
    deg = coeffs.shape[0]
    n = B * T
    nsteps1 = n // _ROWS1
    nsteps2 = n // _ROWS2
    xf = x.reshape(n, D)

    y = pl.pallas_call(
        functools.partial(_fused_kernel, keep, nsteps1),
        grid=(nsteps1 + nsteps2,),
        in_specs=[
            pl.BlockSpec((1, D), lambda i: (0, 0)),
            pl.BlockSpec((D, 1), lambda i: (0, 0)),
            pl.BlockSpec((_ROWS1, D), lambda i: (jnp.minimum(i, nsteps1 - 1), 0)),
            pl.BlockSpec((_ROWS2, D), lambda i: (jnp.maximum(i - nsteps1, 0), 0)),
            pl.BlockSpec(memory_space=pltpu.SMEM),
        ],
        out_specs=pl.BlockSpec((_ROWS2, D), lambda i: (jnp.maximum(i - nsteps1, 0), 0)),
        out_shape=jax.ShapeDtypeStruct((n, D), jnp.float32),
        scratch_shapes=[
            pltpu.VMEM((1, D), jnp.float32),
            pltpu.VMEM((1, D), jnp.float32),
            pltpu.VMEM((deg, D), jnp.float32),
            pltpu.SMEM((1, 1), jnp.float32),
        ],
    )(importance.reshape(1, D), importance.reshape(D, 1), xf, xf,
      coeffs.reshape(1, deg))

    return y.reshape(B, T, D)
